# Initial kernel scaffold; baseline (speedup 1.0000x reference)
import jax
import jax.numpy as jnp
from jax.experimental import pallas as pl


def kernel(x, edge_index, W1_l, W1_r, b1, W2_l, W2_r, b2):
    raise NotImplementedError("write your pallas kernel here")


# speedup vs baseline: 15.9175x; 15.9175x over previous
import functools

import jax
import jax.numpy as jnp
from jax import lax
from jax.experimental import pallas as pl
from jax.experimental.pallas import tpu as pltpu
from jax.experimental.pallas import tpu_sc as plsc

N = 10000
E = 320000
D_IN = 128
D_H = 64
D_OUT = 32

NC = 2
NS = 16
NW = NC * NS
EPW = E // NW
BSZ = 125
NBLK = EPW // BSZ
RPT = N // NS

_SC_MESH = plsc.VectorSubcoreMesh(
    core_axis_name="c", subcore_axis_name="s", num_cores=NC, num_subcores=NS)


def _make_segsum(D, with_counts):
  out_type = [jax.ShapeDtypeStruct((NC, NS, RPT, D), jnp.float32)]
  scratch = [
      pltpu.VMEM((NBLK, BSZ), jnp.int32),
      pltpu.VMEM((NBLK, BSZ), jnp.int32),
      pltpu.VMEM((2, BSZ, D), jnp.float32),
      pltpu.VMEM_SHARED((N, D), jnp.float32),
      pltpu.SemaphoreType.DMA((2,)),
  ]
  if with_counts:
    out_type.append(jax.ShapeDtypeStruct((NC, NS, RPT, 16), jnp.float32))
    scratch.append(pltpu.VMEM((BSZ, 16), jnp.float32))
    scratch.append(pltpu.VMEM_SHARED((N, 16), jnp.float32))

  def body(table, srcb, dstb, zeros_d, *rest):
    if with_counts:
      (zeros16, ones_hbm, out, cnt_out,
       idx_s, idx_d, rows, acc, sems, ones_v, acc_c) = rest
    else:
      out, idx_s, idx_d, rows, acc, sems = rest
    c = lax.axis_index("c")
    s = lax.axis_index("s")
    wid = s * NC + c
    pltpu.sync_copy(srcb.at[wid], idx_s)
    pltpu.sync_copy(dstb.at[wid], idx_d)
    r0 = s * RPT
    pltpu.sync_copy(zeros_d.at[s], acc.at[pl.ds(r0, RPT)])
    if with_counts:
      pltpu.sync_copy(zeros16.at[s], acc_c.at[pl.ds(r0, RPT)])
      pltpu.sync_copy(ones_hbm, ones_v)
    plsc.subcore_barrier()

    pltpu.async_copy(table.at[idx_s.at[0]], rows.at[0], sems.at[0])

    def step(j, carry):
      buf = lax.rem(j, 2)

      @pl.when(j + 1 < NBLK)
      def _():
        pltpu.async_copy(
            table.at[idx_s.at[j + 1]], rows.at[1 - buf], sems.at[1 - buf])

      pltpu.make_async_copy(
          table.at[idx_s.at[j]], rows.at[buf], sems.at[buf]).wait()
      pltpu.sync_copy(rows.at[buf], acc.at[idx_d.at[j]], add=True)
      if with_counts:
        pltpu.sync_copy(ones_v, acc_c.at[idx_d.at[j]], add=True)
      return carry

    lax.fori_loop(0, NBLK, step, 0)
    plsc.subcore_barrier()
    pltpu.sync_copy(acc.at[pl.ds(r0, RPT)], out.at[c, s])
    if with_counts:
      pltpu.sync_copy(acc_c.at[pl.ds(r0, RPT)], cnt_out.at[c, s])

  return pl.kernel(
      body, out_type=tuple(out_type), mesh=_SC_MESH, scratch_types=scratch,
      compiler_params=pltpu.CompilerParams(use_tc_tiling_on_sc=False))


_segsum64 = _make_segsum(D_H, with_counts=True)
_segsum32 = _make_segsum(D_OUT, with_counts=False)

_BN = 2000
_GRID = N // _BN


def _proj1_body(x_ref, wl_ref, wr_ref, b_ref, y_ref, r_ref):
  xb = x_ref[...]
  y_ref[...] = jnp.dot(xb, wl_ref[...], preferred_element_type=jnp.float32)
  r_ref[...] = (jnp.dot(xb, wr_ref[...], preferred_element_type=jnp.float32)
                + b_ref[...])


_proj1 = pl.pallas_call(
    _proj1_body,
    grid=(_GRID,),
    in_specs=[
        pl.BlockSpec((_BN, D_IN), lambda i: (i, 0)),
        pl.BlockSpec((D_IN, D_H), lambda i: (0, 0)),
        pl.BlockSpec((D_IN, D_H), lambda i: (0, 0)),
        pl.BlockSpec((1, D_H), lambda i: (0, 0)),
    ],
    out_specs=[
        pl.BlockSpec((_BN, D_H), lambda i: (i, 0)),
        pl.BlockSpec((_BN, D_H), lambda i: (i, 0)),
    ],
    out_shape=[
        jax.ShapeDtypeStruct((N, D_H), jnp.float32),
        jax.ShapeDtypeStruct((N, D_H), jnp.float32),
    ],
)


def _mid_body(s_ref, c_ref, r_ref, wl_ref, wr_ref, b_ref, y_ref, r2_ref):
  ssum = s_ref[0] + s_ref[1]
  cnt = c_ref[0, :, 0:1] + c_ref[1, :, 0:1]
  h = jnp.maximum(ssum / jnp.maximum(cnt, 1.0) + r_ref[...], 0.0)
  y_ref[...] = jnp.dot(h, wl_ref[...], preferred_element_type=jnp.float32)
  r2_ref[...] = (jnp.dot(h, wr_ref[...], preferred_element_type=jnp.float32)
                 + b_ref[...])


_mid = pl.pallas_call(
    _mid_body,
    grid=(_GRID,),
    in_specs=[
        pl.BlockSpec((NC, _BN, D_H), lambda i: (0, i, 0)),
        pl.BlockSpec((NC, _BN, 16), lambda i: (0, i, 0)),
        pl.BlockSpec((_BN, D_H), lambda i: (i, 0)),
        pl.BlockSpec((D_H, D_OUT), lambda i: (0, 0)),
        pl.BlockSpec((D_H, D_OUT), lambda i: (0, 0)),
        pl.BlockSpec((1, D_OUT), lambda i: (0, 0)),
    ],
    out_specs=[
        pl.BlockSpec((_BN, D_OUT), lambda i: (i, 0)),
        pl.BlockSpec((_BN, D_OUT), lambda i: (i, 0)),
    ],
    out_shape=[
        jax.ShapeDtypeStruct((N, D_OUT), jnp.float32),
        jax.ShapeDtypeStruct((N, D_OUT), jnp.float32),
    ],
)


def _fin_body(s_ref, c_ref, r_ref, o_ref):
  ssum = s_ref[0] + s_ref[1]
  cnt = c_ref[0, :, 0:1] + c_ref[1, :, 0:1]
  o_ref[...] = ssum / jnp.maximum(cnt, 1.0) + r_ref[...]


_fin = pl.pallas_call(
    _fin_body,
    grid=(_GRID,),
    in_specs=[
        pl.BlockSpec((NC, _BN, D_OUT), lambda i: (0, i, 0)),
        pl.BlockSpec((NC, _BN, 16), lambda i: (0, i, 0)),
        pl.BlockSpec((_BN, D_OUT), lambda i: (i, 0)),
    ],
    out_specs=pl.BlockSpec((_BN, D_OUT), lambda i: (i, 0)),
    out_shape=jax.ShapeDtypeStruct((N, D_OUT), jnp.float32),
)


@jax.jit
def _run(x, edge_index, W1_l, W1_r, b1, W2_l, W2_r, b2):
  src = edge_index[0].reshape(NW, NBLK, BSZ)
  dst = edge_index[1].reshape(NW, NBLK, BSZ)
  z64 = jnp.zeros((NS, RPT, D_H), jnp.float32)
  z32 = jnp.zeros((NS, RPT, D_OUT), jnp.float32)
  z16 = jnp.zeros((NS, RPT, 16), jnp.float32)
  ones = jnp.ones((BSZ, 16), jnp.float32)

  y1, r1 = _proj1(x, W1_l, W1_r, b1.reshape(1, D_H))
  sum1, cnt = _segsum64(y1, src, dst, z64, z16, ones)
  sum1 = sum1.reshape(NC, N, D_H)
  cnt = cnt.reshape(NC, N, 16)
  y2, r2 = _mid(sum1, cnt, r1, W2_l, W2_r, b2.reshape(1, D_OUT))
  (sum2,) = _segsum32(y2, src, dst, z32)
  sum2 = sum2.reshape(NC, N, D_OUT)
  return _fin(sum2, cnt, r2)


def kernel(x, edge_index, W1_l, W1_r, b1, W2_l, W2_r, b2):
  return _run(x, edge_index, W1_l, W1_r, b1, W2_l, W2_r, b2)
